# baseline (speedup 1.0000x reference)
import functools

import jax
import jax.numpy as jnp
from jax.experimental import pallas as pl
from jax.experimental.pallas import tpu as pltpu

_KB = 4096


def _center_loss_body(x_ref, t_ref, c_ref, o_ref, acc_ref, *, k_total, kb):
    k = pl.program_id(0)
    nk = pl.num_programs(0)

    @pl.when(k == 0)
    def _init():
        acc_ref[...] = jnp.zeros_like(acc_ref)

    @pl.when(k < nk - 1)
    def _full_block():
        acc_ref[...] += jnp.dot(
            t_ref[...], c_ref[...], preferred_element_type=jnp.float32
        )

    @pl.when(k == nk - 1)
    def _tail_and_epilogue():
        rem = k_total - (nk - 1) * kb
        acc_ref[...] += jnp.dot(
            t_ref[:, :rem], c_ref[:rem, :], preferred_element_type=jnp.float32
        )
        d = x_ref[...] - acc_ref[...]
        o_ref[...] = jnp.sum(d * d, axis=1, keepdims=True)


def kernel(x, targets, centers):
    b, e = x.shape
    k_total = targets.shape[1]
    nk = pl.cdiv(k_total, _KB)

    body = functools.partial(_center_loss_body, k_total=k_total, kb=_KB)
    return pl.pallas_call(
        body,
        grid=(nk,),
        in_specs=[
            pl.BlockSpec((b, e), lambda k: (0, 0)),
            pl.BlockSpec((b, _KB), lambda k: (0, k)),
            pl.BlockSpec((_KB, e), lambda k: (k, 0)),
        ],
        out_specs=pl.BlockSpec((b, 1), lambda k: (0, 0)),
        out_shape=jax.ShapeDtypeStruct((b, 1), jnp.float32),
        scratch_shapes=[pltpu.VMEM((b, e), jnp.float32)],
        compiler_params=pltpu.CompilerParams(
            dimension_semantics=("arbitrary",),
        ),
    )(x, targets, centers)
